# baseline (speedup 1.0000x reference)
import functools
import math

import jax
import jax.numpy as jnp
from jax import lax
from jax.experimental import pallas as pl
from jax.experimental.pallas import tpu as pltpu
from jax.experimental.pallas import tpu_sc as plsc

_N = 10000
_E = 320000
_F = 128
_S = 10
_R = 8
_EPS = 1.0 / math.sqrt(32.0)

_CH = 80
_NCHUNKS = _E // _CH
_NW = 32
_CPW = _NCHUNKS // _NW
_FCH = 80
_NFL = _N // _FCH
_NFL_FULL = _NFL // 16
_NFL_REM = _NFL % 16


def _mlp_body(re_ref, w1_ref, w2_ref, w3_ref, w4_ref, out_ref):
    h = jnp.dot(re_ref[...], w1_ref[...], preferred_element_type=jnp.float32)
    h = h * (1.0 / math.sqrt(float(_R)))
    h = h * jax.nn.sigmoid(h)
    h = jnp.dot(h, w2_ref[...], preferred_element_type=jnp.float32) * 0.125
    h = h * jax.nn.sigmoid(h)
    h = jnp.dot(h, w3_ref[...], preferred_element_type=jnp.float32) * 0.125
    h = h * jax.nn.sigmoid(h)
    out_ref[...] = jnp.dot(h, w4_ref[...], preferred_element_type=jnp.float32) * (
        0.125 * _EPS)


def _edge_mlp(re, w1, w2, w3, w4):
    be = 4000
    return pl.pallas_call(
        _mlp_body,
        grid=(_E // be,),
        in_specs=[
            pl.BlockSpec((be, _R), lambda i: (i, 0)),
            pl.BlockSpec((_R, 64), lambda i: (0, 0)),
            pl.BlockSpec((64, 64), lambda i: (0, 0)),
            pl.BlockSpec((64, 64), lambda i: (0, 0)),
            pl.BlockSpec((64, _F), lambda i: (0, 0)),
        ],
        out_specs=pl.BlockSpec((be, _F), lambda i: (i, 0)),
        out_shape=jax.ShapeDtypeStruct((_E, _F), jnp.float32),
    )(re, w1, w2, w3, w4)


def _lin_up_body(nf_ref, w_ref, out_ref):
    out_ref[...] = jnp.dot(
        nf_ref[...], w_ref[...], preferred_element_type=jnp.float32
    ) * (1.0 / math.sqrt(float(_F)))


def _lin_up(nf, w):
    bn = 2000
    return pl.pallas_call(
        _lin_up_body,
        grid=(_N // bn,),
        in_specs=[
            pl.BlockSpec((bn, _F), lambda i: (i, 0)),
            pl.BlockSpec((_F, _F), lambda i: (0, 0)),
        ],
        out_specs=pl.BlockSpec((bn, _F), lambda i: (i, 0)),
        out_shape=jax.ShapeDtypeStruct((_N, _F), jnp.float32),
    )(nf, w)


def _sc_agg_body(x_h, mix_h, snd_h, rcv_h, out_h,
                 snd_v, rcv_v, xr_v, mx_v, agg_sh, isem, gsem, ssem):
    cid = lax.axis_index("c")
    sid = lax.axis_index("s")
    w = sid * 2 + cid

    zv = jnp.zeros((16,), jnp.float32)

    def _zero_row(i, carry):
        for j in range(8):
            xr_v[0, i, pl.ds(j * 16, 16)] = zv
        return carry

    lax.fori_loop(0, _FCH, _zero_row, 0)

    nfl = jnp.where(sid < _NFL_REM, _NFL_FULL + 1, _NFL_FULL)

    def _zero_chunk(j, carry):
        r0 = (sid + j * 16) * _FCH
        pltpu.sync_copy(xr_v.at[0, pl.ds(0, _FCH)], agg_sh.at[pl.ds(r0, _FCH)])
        return carry

    lax.fori_loop(0, nfl, _zero_chunk, 0)
    plsc.subcore_barrier()

    start = _CPW * w

    pltpu.sync_copy(snd_h.at[start], snd_v.at[0])
    pltpu.sync_copy(rcv_h.at[start], rcv_v.at[0])
    pltpu.async_copy(snd_h.at[start + 1], snd_v.at[1], isem)
    pltpu.async_copy(rcv_h.at[start + 1], rcv_v.at[1], isem)
    pltpu.async_copy(x_h.at[snd_v.at[0]], xr_v.at[0], gsem)
    pltpu.async_copy(mix_h.at[start], mx_v.at[0], gsem)

    def _chunk(i, carry):
        b = lax.rem(i, 2)
        nb = 1 - b
        s4 = lax.rem(i, 4)

        @pl.when(i > 0)
        def _wait_prev_scatter():
            pltpu.make_async_copy(mix_h.at[start], mx_v.at[0], ssem).wait()

        pltpu.make_async_copy(x_h.at[snd_v.at[0]], xr_v.at[0], gsem).wait()
        pltpu.make_async_copy(mix_h.at[start], mx_v.at[0], gsem).wait()

        @pl.when(i + 1 < _CPW)
        def _issue_next():
            s41 = lax.rem(i + 1, 4)
            pltpu.make_async_copy(snd_h.at[start], snd_v.at[0], isem).wait()
            pltpu.make_async_copy(rcv_h.at[start], rcv_v.at[0], isem).wait()
            pltpu.async_copy(x_h.at[snd_v.at[s41]], xr_v.at[nb], gsem)
            pltpu.async_copy(mix_h.at[start + i + 1], mx_v.at[nb], gsem)

        @pl.when(i + 2 < _CPW)
        def _prefetch_idx():
            s42 = lax.rem(i + 2, 4)
            pltpu.async_copy(snd_h.at[start + i + 2], snd_v.at[s42], isem)
            pltpu.async_copy(rcv_h.at[start + i + 2], rcv_v.at[s42], isem)

        def _mul(e, c2):
            for j in range(8):
                sl = pl.ds(j * 16, 16)
                mx_v[b, e, sl] = mx_v[b, e, sl] * xr_v[b, e, sl]
            return c2

        lax.fori_loop(0, _CH, _mul, 0)
        pltpu.async_copy(mx_v.at[b], agg_sh.at[rcv_v.at[s4]], ssem, add=True)
        return carry

    lax.fori_loop(0, _CPW, _chunk, 0)
    pltpu.make_async_copy(mix_h.at[start], mx_v.at[0], ssem).wait()
    plsc.subcore_barrier()

    def _flush_chunk(j, carry):
        r0 = (sid + j * 16) * _FCH
        pltpu.sync_copy(agg_sh.at[pl.ds(r0, _FCH)], xr_v.at[0, pl.ds(0, _FCH)])
        pltpu.sync_copy(xr_v.at[0, pl.ds(0, _FCH)], out_h.at[cid, pl.ds(r0, _FCH)])
        return carry

    lax.fori_loop(0, nfl, _flush_chunk, 0)


def _sc_aggregate(x, mix3, snd2, rcv2):
    mesh = plsc.VectorSubcoreMesh(core_axis_name="c", subcore_axis_name="s")
    fn = functools.partial(
        pl.kernel,
        mesh=mesh,
        out_type=jax.ShapeDtypeStruct((2, _N, _F), jnp.float32),
        scratch_types=[
            pltpu.VMEM((4, _CH), jnp.int32),
            pltpu.VMEM((4, _CH), jnp.int32),
            pltpu.VMEM((2, _CH, _F), jnp.float32),
            pltpu.VMEM((2, _CH, _F), jnp.float32),
            pltpu.VMEM_SHARED((_N, _F), jnp.float32),
            pltpu.SemaphoreType.DMA,
            pltpu.SemaphoreType.DMA,
            pltpu.SemaphoreType.DMA,
        ],
    )(_sc_agg_body)
    return fn(x, mix3, snd2, rcv2)


def _post_body(aggp_ref, nf_ref, wld_ref, wsym_ref, wlp_ref, wskip_ref,
               wrm_ref, wr_ref, out1_ref, out2_ref):
    agg = aggp_ref[0] + aggp_ref[1]
    x2 = jnp.dot(agg, wld_ref[...], preferred_element_type=jnp.float32) * (
        1.0 / math.sqrt(float(_F)))
    w0 = wsym_ref[0, 0:1, :]
    w1 = wsym_ref[0, 1:2, :]
    w2 = wsym_ref[0, 2:3, :]
    x3 = x2 * (w0 + x2 * (w1 + x2 * w2))
    sc = jnp.dot(nf_ref[...], wskip_ref[0], preferred_element_type=jnp.float32) * (
        1.0 / math.sqrt(float(_F * _S)))
    x4 = jnp.dot(x3, wlp_ref[...], preferred_element_type=jnp.float32) * (
        1.0 / math.sqrt(float(_F))) + sc
    out2_ref[...] = x4
    h = jnp.dot(x4, wrm_ref[...], preferred_element_type=jnp.float32) * (
        1.0 / math.sqrt(float(_F)))
    h = h * jax.nn.sigmoid(h)
    out1_ref[...] = jnp.dot(h, wr_ref[...], preferred_element_type=jnp.float32) * 0.25


def _post(aggp, nf, wld, wsym, wlp, wskip, wrm, wr):
    bn = _N // _S
    return pl.pallas_call(
        _post_body,
        grid=(_S,),
        in_specs=[
            pl.BlockSpec((2, bn, _F), lambda s: (0, s, 0)),
            pl.BlockSpec((bn, _F), lambda s: (s, 0)),
            pl.BlockSpec((_F, _F), lambda s: (0, 0)),
            pl.BlockSpec((1, 3, _F), lambda s: (s, 0, 0)),
            pl.BlockSpec((_F, _F), lambda s: (0, 0)),
            pl.BlockSpec((1, _F, _F), lambda s: (s, 0, 0)),
            pl.BlockSpec((_F, 16), lambda s: (0, 0)),
            pl.BlockSpec((16, 1), lambda s: (0, 0)),
        ],
        out_specs=[
            pl.BlockSpec((bn, 1), lambda s: (s, 0)),
            pl.BlockSpec((bn, _F), lambda s: (s, 0)),
        ],
        out_shape=[
            jax.ShapeDtypeStruct((_N, 1), jnp.float32),
            jax.ShapeDtypeStruct((_N, _F), jnp.float32),
        ],
    )(aggp, nf, wld, wsym, wlp, wskip, wrm, wr)


def kernel(vectors, node_feats, num_species, radial_embeddings, senders,
           receivers, w_lin_up, mlp_w1, mlp_w2, mlp_w3, mlp_w4, w_lin_down,
           w_sym, w_lin_post, w_skip, w_readout_mlp, w_readout):
    mix = _edge_mlp(radial_embeddings, mlp_w1, mlp_w2, mlp_w3, mlp_w4)
    x = _lin_up(node_feats, w_lin_up)
    aggp = _sc_aggregate(
        x,
        mix.reshape(_NCHUNKS, _CH, _F),
        senders.reshape(_NCHUNKS, _CH),
        receivers.reshape(_NCHUNKS, _CH),
    )
    node_outputs, node_feats_out = _post(
        aggp, node_feats, w_lin_down, w_sym, w_lin_post, w_skip,
        w_readout_mlp, w_readout)
    return (node_outputs, node_feats_out)


# speedup vs baseline: 1.6406x; 1.6406x over previous
import functools
import math

import jax
import jax.numpy as jnp
from jax import lax
from jax.experimental import pallas as pl
from jax.experimental.pallas import tpu as pltpu
from jax.experimental.pallas import tpu_sc as plsc

_N = 10000
_E = 320000
_F = 128
_S = 10
_R = 8
_EPS = 1.0 / math.sqrt(32.0)

_CH = 80
_NCHUNKS = _E // _CH
_NW = 32
_CPW = _NCHUNKS // _NW
_FCH = 80
_NFL = _N // _FCH
_NFL_FULL = _NFL // 16
_NFL_REM = _NFL % 16


def _mlp_body(re_ref, w1_ref, w2_ref, w3_ref, w4_ref, out_ref):
    h = jnp.dot(re_ref[...], w1_ref[...], preferred_element_type=jnp.float32)
    h = h * (1.0 / math.sqrt(float(_R)))
    h = h * jax.nn.sigmoid(h)
    h = jnp.dot(h, w2_ref[...], preferred_element_type=jnp.float32) * 0.125
    h = h * jax.nn.sigmoid(h)
    h = jnp.dot(h, w3_ref[...], preferred_element_type=jnp.float32) * 0.125
    h = h * jax.nn.sigmoid(h)
    out_ref[...] = jnp.dot(h, w4_ref[...], preferred_element_type=jnp.float32) * (
        0.125 * _EPS)


def _edge_mlp(re, w1, w2, w3, w4):
    be = 4000
    return pl.pallas_call(
        _mlp_body,
        grid=(_E // be,),
        in_specs=[
            pl.BlockSpec((be, _R), lambda i: (i, 0)),
            pl.BlockSpec((_R, 64), lambda i: (0, 0)),
            pl.BlockSpec((64, 64), lambda i: (0, 0)),
            pl.BlockSpec((64, 64), lambda i: (0, 0)),
            pl.BlockSpec((64, _F), lambda i: (0, 0)),
        ],
        out_specs=pl.BlockSpec((be, _F), lambda i: (i, 0)),
        out_shape=jax.ShapeDtypeStruct((_E, _F), jnp.float32),
    )(re, w1, w2, w3, w4)


def _lin_up_body(nf_ref, w_ref, out_ref):
    out_ref[...] = jnp.dot(
        nf_ref[...], w_ref[...], preferred_element_type=jnp.float32
    ) * (1.0 / math.sqrt(float(_F)))


def _lin_up(nf, w):
    bn = 2000
    return pl.pallas_call(
        _lin_up_body,
        grid=(_N // bn,),
        in_specs=[
            pl.BlockSpec((bn, _F), lambda i: (i, 0)),
            pl.BlockSpec((_F, _F), lambda i: (0, 0)),
        ],
        out_specs=pl.BlockSpec((bn, _F), lambda i: (i, 0)),
        out_shape=jax.ShapeDtypeStruct((_N, _F), jnp.float32),
    )(nf, w)


def _sc_agg_body(x_h, mix_h, snd_h, rcv_h, out_h,
                 snd_v, rcv_v, xr_v, mx_v, agg_sh, isem, gsem, ssem):
    cid = lax.axis_index("c")
    sid = lax.axis_index("s")
    w = sid * 2 + cid

    zv = jnp.zeros((16,), jnp.float32)

    def _zero_row(i, carry):
        for j in range(8):
            xr_v[0, i, pl.ds(j * 16, 16)] = zv
        return carry

    lax.fori_loop(0, _FCH, _zero_row, 0)

    nfl = jnp.where(sid < _NFL_REM, _NFL_FULL + 1, _NFL_FULL)

    def _zero_chunk(j, carry):
        r0 = (sid + j * 16) * _FCH
        pltpu.sync_copy(xr_v.at[0, pl.ds(0, _FCH)], agg_sh.at[pl.ds(r0, _FCH)])
        return carry

    lax.fori_loop(0, nfl, _zero_chunk, 0)
    plsc.subcore_barrier()

    start = _CPW * w

    pltpu.sync_copy(snd_h.at[start], snd_v.at[0])
    pltpu.sync_copy(rcv_h.at[start], rcv_v.at[0])
    pltpu.async_copy(snd_h.at[start + 1], snd_v.at[1], isem)
    pltpu.async_copy(rcv_h.at[start + 1], rcv_v.at[1], isem)
    pltpu.async_copy(x_h.at[snd_v.at[0]], xr_v.at[0], gsem)
    pltpu.async_copy(mix_h.at[start], mx_v.at[0], gsem)

    def _chunk(i, carry):
        b = lax.rem(i, 2)
        nb = 1 - b
        s4 = lax.rem(i, 4)

        @pl.when(i > 0)
        def _wait_prev_scatter():
            pltpu.make_async_copy(mix_h.at[start], mx_v.at[0], ssem).wait()

        pltpu.make_async_copy(x_h.at[snd_v.at[0]], xr_v.at[0], gsem).wait()
        pltpu.make_async_copy(mix_h.at[start], mx_v.at[0], gsem).wait()

        @pl.when(i + 1 < _CPW)
        def _issue_next():
            s41 = lax.rem(i + 1, 4)
            pltpu.make_async_copy(snd_h.at[start], snd_v.at[0], isem).wait()
            pltpu.make_async_copy(rcv_h.at[start], rcv_v.at[0], isem).wait()
            pltpu.async_copy(x_h.at[snd_v.at[s41]], xr_v.at[nb], gsem)
            pltpu.async_copy(mix_h.at[start + i + 1], mx_v.at[nb], gsem)

        @pl.when(i + 2 < _CPW)
        def _prefetch_idx():
            s42 = lax.rem(i + 2, 4)
            pltpu.async_copy(snd_h.at[start + i + 2], snd_v.at[s42], isem)
            pltpu.async_copy(rcv_h.at[start + i + 2], rcv_v.at[s42], isem)

        @functools.partial(plsc.parallel_loop, 0, _CH, unroll=4)
        def _mul(e):
            for j in range(8):
                sl = pl.ds(j * 16, 16)
                mx_v[b, e, sl] = mx_v[b, e, sl] * xr_v[b, e, sl]

        pltpu.async_copy(mx_v.at[b], agg_sh.at[rcv_v.at[s4]], ssem, add=True)
        return carry

    lax.fori_loop(0, _CPW, _chunk, 0)
    pltpu.make_async_copy(mix_h.at[start], mx_v.at[0], ssem).wait()
    plsc.subcore_barrier()

    def _flush_chunk(j, carry):
        r0 = (sid + j * 16) * _FCH
        pltpu.sync_copy(agg_sh.at[pl.ds(r0, _FCH)], xr_v.at[0, pl.ds(0, _FCH)])
        pltpu.sync_copy(xr_v.at[0, pl.ds(0, _FCH)], out_h.at[cid, pl.ds(r0, _FCH)])
        return carry

    lax.fori_loop(0, nfl, _flush_chunk, 0)


def _sc_aggregate(x, mix3, snd2, rcv2):
    mesh = plsc.VectorSubcoreMesh(core_axis_name="c", subcore_axis_name="s")
    fn = functools.partial(
        pl.kernel,
        mesh=mesh,
        out_type=jax.ShapeDtypeStruct((2, _N, _F), jnp.float32),
        scratch_types=[
            pltpu.VMEM((4, _CH), jnp.int32),
            pltpu.VMEM((4, _CH), jnp.int32),
            pltpu.VMEM((2, _CH, _F), jnp.float32),
            pltpu.VMEM((2, _CH, _F), jnp.float32),
            pltpu.VMEM_SHARED((_N, _F), jnp.float32),
            pltpu.SemaphoreType.DMA,
            pltpu.SemaphoreType.DMA,
            pltpu.SemaphoreType.DMA,
        ],
    )(_sc_agg_body)
    return fn(x, mix3, snd2, rcv2)


def _post_body(aggp_ref, nf_ref, wld_ref, wsym_ref, wlp_ref, wskip_ref,
               wrm_ref, wr_ref, out1_ref, out2_ref):
    agg = aggp_ref[0] + aggp_ref[1]
    x2 = jnp.dot(agg, wld_ref[...], preferred_element_type=jnp.float32) * (
        1.0 / math.sqrt(float(_F)))
    w0 = wsym_ref[0, 0:1, :]
    w1 = wsym_ref[0, 1:2, :]
    w2 = wsym_ref[0, 2:3, :]
    x3 = x2 * (w0 + x2 * (w1 + x2 * w2))
    sc = jnp.dot(nf_ref[...], wskip_ref[0], preferred_element_type=jnp.float32) * (
        1.0 / math.sqrt(float(_F * _S)))
    x4 = jnp.dot(x3, wlp_ref[...], preferred_element_type=jnp.float32) * (
        1.0 / math.sqrt(float(_F))) + sc
    out2_ref[...] = x4
    h = jnp.dot(x4, wrm_ref[...], preferred_element_type=jnp.float32) * (
        1.0 / math.sqrt(float(_F)))
    h = h * jax.nn.sigmoid(h)
    out1_ref[...] = jnp.dot(h, wr_ref[...], preferred_element_type=jnp.float32) * 0.25


def _post(aggp, nf, wld, wsym, wlp, wskip, wrm, wr):
    bn = _N // _S
    return pl.pallas_call(
        _post_body,
        grid=(_S,),
        in_specs=[
            pl.BlockSpec((2, bn, _F), lambda s: (0, s, 0)),
            pl.BlockSpec((bn, _F), lambda s: (s, 0)),
            pl.BlockSpec((_F, _F), lambda s: (0, 0)),
            pl.BlockSpec((1, 3, _F), lambda s: (s, 0, 0)),
            pl.BlockSpec((_F, _F), lambda s: (0, 0)),
            pl.BlockSpec((1, _F, _F), lambda s: (s, 0, 0)),
            pl.BlockSpec((_F, 16), lambda s: (0, 0)),
            pl.BlockSpec((16, 1), lambda s: (0, 0)),
        ],
        out_specs=[
            pl.BlockSpec((bn, 1), lambda s: (s, 0)),
            pl.BlockSpec((bn, _F), lambda s: (s, 0)),
        ],
        out_shape=[
            jax.ShapeDtypeStruct((_N, 1), jnp.float32),
            jax.ShapeDtypeStruct((_N, _F), jnp.float32),
        ],
    )(aggp, nf, wld, wsym, wlp, wskip, wrm, wr)


def kernel(vectors, node_feats, num_species, radial_embeddings, senders,
           receivers, w_lin_up, mlp_w1, mlp_w2, mlp_w3, mlp_w4, w_lin_down,
           w_sym, w_lin_post, w_skip, w_readout_mlp, w_readout):
    mix = _edge_mlp(radial_embeddings, mlp_w1, mlp_w2, mlp_w3, mlp_w4)
    x = _lin_up(node_feats, w_lin_up)
    aggp = _sc_aggregate(
        x,
        mix.reshape(_NCHUNKS, _CH, _F),
        senders.reshape(_NCHUNKS, _CH),
        receivers.reshape(_NCHUNKS, _CH),
    )
    node_outputs, node_feats_out = _post(
        aggp, node_feats, w_lin_down, w_sym, w_lin_post, w_skip,
        w_readout_mlp, w_readout)
    return (node_outputs, node_feats_out)
